# baseline (speedup 1.0000x reference)
import functools

import jax
import jax.numpy as jnp
from jax.experimental import pallas as pl
from jax.experimental.pallas import tpu as pltpu

E = 8
D = 2048
DFF = 4096
T = 256
BF = 1024
NF = DFF // BF


def _moe_kernel(xT_ref, wr_ref, br_ref, w1_ref, b1_ref, w2_ref, b2_ref,
                w3_ref, b3_ref, out_ref, wrow_ref):
    e = pl.program_id(0)
    f = pl.program_id(1)

    @pl.when((e == 0) & (f == 0))
    def _init():
        logits = jnp.dot(wr_ref[...].astype(jnp.bfloat16),
                         xT_ref[...].astype(jnp.bfloat16),
                         preferred_element_type=jnp.float32) + br_ref[...]
        idx = jax.lax.broadcasted_iota(jnp.int32, (E, T), 0)
        m1 = jnp.max(logits, axis=0, keepdims=True)
        i1 = jnp.min(jnp.where(logits == m1, idx, E), axis=0, keepdims=True)
        sel1 = idx == i1
        masked = jnp.where(sel1, -jnp.inf, logits)
        m2 = jnp.max(masked, axis=0, keepdims=True)
        i2 = jnp.min(jnp.where(masked == m2, idx, E), axis=0, keepdims=True)
        sel2 = idx == i2
        p1 = 1.0 / (1.0 + jnp.exp(m2 - m1))
        wrow_ref[...] = jnp.where(sel1, p1, 0.0) + jnp.where(sel2, 1.0 - p1, 0.0)
        out_ref[...] = jnp.zeros_like(out_ref)

    xv = xT_ref[...]
    h1 = jnp.dot(w1_ref[0], xv, preferred_element_type=jnp.float32) + b1_ref[0]
    h2 = jnp.dot(w2_ref[0], xv, preferred_element_type=jnp.float32) + b2_ref[0]
    h = h2 * (h1 * jax.nn.sigmoid(h1))
    yT = jnp.dot(w3_ref[0], h, preferred_element_type=jnp.float32)
    wrow = wrow_ref[pl.ds(e, 1), :]
    acc = yT * wrow

    @pl.when(f == 0)
    def _bias3():
        out_ref[...] += b3_ref[0] * wrow

    out_ref[...] += acc


@functools.partial(jax.jit, static_argnames=())
def kernel(x, Wr, br, W1, b1, W2, b2, W3, b3):
    b, s, d = x.shape
    xT = x.reshape(b * s, d).T
    outT = pl.pallas_call(
        _moe_kernel,
        grid=(E, NF),
        in_specs=[
            pl.BlockSpec((D, T), lambda e, f: (0, 0)),
            pl.BlockSpec((E, D), lambda e, f: (0, 0)),
            pl.BlockSpec((E, 1), lambda e, f: (0, 0)),
            pl.BlockSpec((1, BF, D), lambda e, f: (e, f, 0)),
            pl.BlockSpec((1, BF, 1), lambda e, f: (e, f, 0)),
            pl.BlockSpec((1, BF, D), lambda e, f: (e, f, 0)),
            pl.BlockSpec((1, BF, 1), lambda e, f: (e, f, 0)),
            pl.BlockSpec((1, D, BF), lambda e, f: (e, 0, f)),
            pl.BlockSpec((1, D, 1), lambda e, f: (e, 0, 0)),
        ],
        out_specs=pl.BlockSpec((D, T), lambda e, f: (0, 0)),
        out_shape=jax.ShapeDtypeStruct((D, T), jnp.float32),
        scratch_shapes=[
            pltpu.VMEM((E, T), jnp.float32),
        ],
    )(xT, Wr, br.reshape(E, 1), W1, b1.reshape(E, DFF, 1),
      W2, b2.reshape(E, DFF, 1), W3, b3.reshape(E, D, 1))
    return outT.T.reshape(b, s, d)


# speedup vs baseline: 1.0207x; 1.0207x over previous
import jax
import jax.numpy as jnp
from jax.experimental import pallas as pl
from jax.experimental.pallas import tpu as pltpu

E = 8
D = 2048
DFF = 4096
T = 256
BF = 512
HBF = BF // 2
NF = DFF // BF


def _moe_kernel(xT_ref, wr_ref, br_ref, w1a_ref, w1b_ref, w2a_ref, w2b_ref,
                w3a_ref, w3b_ref, b1_ref, b2_ref, b3_ref,
                out_ref, wrow_ref):
    e = pl.program_id(0)
    f = pl.program_id(1)

    @pl.when((e == 0) & (f == 0))
    def _init():
        logits = jnp.dot(wr_ref[...], xT_ref[...],
                         preferred_element_type=jnp.float32) + br_ref[...]
        idx = jax.lax.broadcasted_iota(jnp.int32, (E, T), 0)
        m1 = jnp.max(logits, axis=0, keepdims=True)
        i1 = jnp.min(jnp.where(logits == m1, idx, E), axis=0, keepdims=True)
        sel1 = idx == i1
        masked = jnp.where(sel1, -jnp.inf, logits)
        m2 = jnp.max(masked, axis=0, keepdims=True)
        i2 = jnp.min(jnp.where(masked == m2, idx, E), axis=0, keepdims=True)
        sel2 = idx == i2
        p1 = 1.0 / (1.0 + jnp.exp(m2 - m1))
        wrow_ref[...] = jnp.where(sel1, p1, 0.0) + jnp.where(sel2, 1.0 - p1, 0.0)
        out_ref[...] = jnp.zeros_like(out_ref)

    xv = xT_ref[...]
    b1a = b1_ref[0, pl.ds(f * BF, HBF), :]
    b1b = b1_ref[0, pl.ds(f * BF + HBF, HBF), :]
    b2a = b2_ref[0, pl.ds(f * BF, HBF), :]
    b2b = b2_ref[0, pl.ds(f * BF + HBF, HBF), :]
    h1a = jnp.dot(w1a_ref[0], xv, preferred_element_type=jnp.float32) + b1a
    h1b = jnp.dot(w1b_ref[0], xv, preferred_element_type=jnp.float32) + b1b
    h2a = jnp.dot(w2a_ref[0], xv, preferred_element_type=jnp.float32) + b2a
    h2b = jnp.dot(w2b_ref[0], xv, preferred_element_type=jnp.float32) + b2b
    ha = h2a * (h1a * jax.nn.sigmoid(h1a))
    hb = h2b * (h1b * jax.nn.sigmoid(h1b))
    yT = (jnp.dot(w3a_ref[0], ha, preferred_element_type=jnp.float32)
          + jnp.dot(w3b_ref[0], hb, preferred_element_type=jnp.float32))
    wrow = wrow_ref[pl.ds(e, 1), :]
    acc = yT * wrow

    @pl.when(f == 0)
    def _bias3():
        out_ref[...] += b3_ref[0] * wrow

    out_ref[...] += acc


def kernel(x, Wr, br, W1, b1, W2, b2, W3, b3):
    b, s, d = x.shape
    xT = x.reshape(b * s, d).T
    outT = pl.pallas_call(
        _moe_kernel,
        grid=(E, NF),
        in_specs=[
            pl.BlockSpec((D, T), lambda e, f: (0, 0)),
            pl.BlockSpec((E, D), lambda e, f: (0, 0)),
            pl.BlockSpec((E, 1), lambda e, f: (0, 0)),
            pl.BlockSpec((1, HBF, D), lambda e, f: (e, 2 * f, 0)),
            pl.BlockSpec((1, HBF, D), lambda e, f: (e, 2 * f + 1, 0)),
            pl.BlockSpec((1, HBF, D), lambda e, f: (e, 2 * f, 0)),
            pl.BlockSpec((1, HBF, D), lambda e, f: (e, 2 * f + 1, 0)),
            pl.BlockSpec((1, D, HBF), lambda e, f: (e, 0, 2 * f)),
            pl.BlockSpec((1, D, HBF), lambda e, f: (e, 0, 2 * f + 1)),
            pl.BlockSpec((1, DFF, 1), lambda e, f: (e, 0, 0)),
            pl.BlockSpec((1, DFF, 1), lambda e, f: (e, 0, 0)),
            pl.BlockSpec((1, D, 1), lambda e, f: (e, 0, 0)),
        ],
        out_specs=pl.BlockSpec((D, T), lambda e, f: (0, 0)),
        out_shape=jax.ShapeDtypeStruct((D, T), jnp.float32),
        scratch_shapes=[
            pltpu.VMEM((E, T), jnp.float32),
        ],
    )(xT, Wr, br.reshape(E, 1), W1, W1, W2, W2, W3, W3,
      b1.reshape(E, DFF, 1), b2.reshape(E, DFF, 1), b3.reshape(E, D, 1))
    return outT.T.reshape(b, s, d)
